# baseline (speedup 1.0000x reference)
import jax
import jax.numpy as jnp
from jax.experimental import pallas as pl
from jax.experimental.pallas import tpu as pltpu

ALPHA = 0.8
GAMMA = 0.8

_LANES = 512
_AUX = 128
_BR = 256


def _roll(x, shift):
    return pltpu.roll(x, (-shift) % x.shape[1], axis=1)


def _update_kernel(a_ref, b_ref, p_ref, q_ref, o_ref):
    q = q_ref[:]
    br = q.shape[0]
    a = a_ref[:]
    b = b_ref[:]
    p = p_ref[:]

    def expand(x):
        return jnp.broadcast_to(x[:, :, None], (br, _AUX, 4)).reshape(br, _LANES)

    a4 = expand(a)
    b4 = expand(b)
    p4 = expand(p)

    c = jax.lax.broadcasted_iota(jnp.int32, (br, _LANES), 1) % 4
    qnext = _roll(q, 1)
    qprev = _roll(q, -1)
    pairmax = jnp.maximum(q, jnp.where(c % 2 == 0, qnext, qprev))
    other = jnp.where(c < 2, _roll(pairmax, 2), _roll(pairmax, -2))
    m0 = jnp.where(c < 2, pairmax, other)
    m1 = jnp.where(c < 2, other, pairmax)
    maxv = jnp.where(b4 == 0, m0, m1)

    upd = q + ALPHA * (p4 + GAMMA * maxv - q)
    o_ref[:] = jnp.where(c == 2 * a4 + b4, upd, q)


def kernel(type_t_matrix, type_t1_matrix, Q_tensor, profit_matrix):
    n = Q_tensor.shape[0]
    rows = n // _AUX
    qv = Q_tensor.reshape(rows, _LANES)
    av = type_t_matrix.reshape(rows, _AUX)
    bv = type_t1_matrix.reshape(rows, _AUX)
    pv = profit_matrix.reshape(rows, _AUX)

    out = pl.pallas_call(
        _update_kernel,
        grid=(rows // _BR,),
        in_specs=[
            pl.BlockSpec((_BR, _AUX), lambda i: (i, 0)),
            pl.BlockSpec((_BR, _AUX), lambda i: (i, 0)),
            pl.BlockSpec((_BR, _AUX), lambda i: (i, 0)),
            pl.BlockSpec((_BR, _LANES), lambda i: (i, 0)),
        ],
        out_specs=pl.BlockSpec((_BR, _LANES), lambda i: (i, 0)),
        out_shape=jax.ShapeDtypeStruct((rows, _LANES), jnp.float32),
        compiler_params=pltpu.CompilerParams(
            dimension_semantics=("arbitrary",),
        ),
    )(av, bv, pv, qv)
    return out.reshape(Q_tensor.shape)


# speedup vs baseline: 1.1188x; 1.1188x over previous
import jax
import jax.numpy as jnp
from jax.experimental import pallas as pl
from jax.experimental.pallas import tpu as pltpu

ALPHA = 0.8
GAMMA = 0.8

_LANES = 512
_AUX = 128
_BR = 256


def _roll(x, shift):
    return pltpu.roll(x, (-shift) % x.shape[1], axis=1)


def _update_kernel(a_ref, b_ref, p_ref, q_ref, o_ref):
    q = q_ref[:]
    br = q.shape[0]
    a = a_ref[:].astype(jnp.float32)
    b = b_ref[:].astype(jnp.float32)
    p = p_ref[:]

    rows_i = jax.lax.broadcasted_iota(jnp.int32, (_AUX, _LANES), 0)
    cols_i = jax.lax.broadcasted_iota(jnp.int32, (_AUX, _LANES), 1)
    e = (cols_i // 4 == rows_i).astype(jnp.float32)
    x3 = jnp.concatenate([b, 2.0 * a + b, p], axis=0)
    y = jax.lax.dot(x3, e, precision=jax.lax.Precision.HIGHEST)
    b4 = y[:br]
    t4 = y[br:2 * br]
    p4 = y[2 * br:]

    c = jax.lax.broadcasted_iota(jnp.int32, (br, _LANES), 1) % 4
    qnext = _roll(q, 1)
    qprev = _roll(q, -1)
    pairmax = jnp.maximum(q, jnp.where(c % 2 == 0, qnext, qprev))
    other = jnp.where(c < 2, _roll(pairmax, 2), _roll(pairmax, -2))
    m0 = jnp.where(c < 2, pairmax, other)
    m1 = jnp.where(c < 2, other, pairmax)
    maxv = jnp.where(b4 == 0.0, m0, m1)

    upd = q + ALPHA * (p4 + GAMMA * maxv - q)
    o_ref[:] = jnp.where(c.astype(jnp.float32) == t4, upd, q)


def kernel(type_t_matrix, type_t1_matrix, Q_tensor, profit_matrix):
    n = Q_tensor.shape[0]
    rows = n // _AUX
    qv = Q_tensor.reshape(rows, _LANES)
    av = type_t_matrix.reshape(rows, _AUX)
    bv = type_t1_matrix.reshape(rows, _AUX)
    pv = profit_matrix.reshape(rows, _AUX)

    out = pl.pallas_call(
        _update_kernel,
        grid=(rows // _BR,),
        in_specs=[
            pl.BlockSpec((_BR, _AUX), lambda i: (i, 0)),
            pl.BlockSpec((_BR, _AUX), lambda i: (i, 0)),
            pl.BlockSpec((_BR, _AUX), lambda i: (i, 0)),
            pl.BlockSpec((_BR, _LANES), lambda i: (i, 0)),
        ],
        out_specs=pl.BlockSpec((_BR, _LANES), lambda i: (i, 0)),
        out_shape=jax.ShapeDtypeStruct((rows, _LANES), jnp.float32),
        compiler_params=pltpu.CompilerParams(
            dimension_semantics=("arbitrary",),
        ),
    )(av, bv, pv, qv)
    return out.reshape(Q_tensor.shape)


# speedup vs baseline: 160.2306x; 143.2200x over previous
import jax
import jax.numpy as jnp
from jax.experimental import pallas as pl
from jax.experimental.pallas import tpu as pltpu

ALPHA = 0.8
GAMMA = 0.8

_L = 2048
_BA = 16
_CHUNK = _BA * _L
_KB = _CHUNK // 128
_MB = 2 * _KB


def _sub_roll(x, shift):
    return pltpu.roll(x, (-shift) % x.shape[0], axis=0)


def _update_kernel(a_ref, b_ref, p_ref, q_ref, o_ref):
    q = q_ref[:]
    q0 = q[0]
    q1 = q[1]

    def aux2(x):
        xk = x.reshape(_KB, 128)
        return jnp.broadcast_to(xk[:, None, :], (_KB, 2, 128)).reshape(_MB, 128)

    a2 = aux2(a_ref[:]).astype(jnp.float32)
    b2 = aux2(b_ref[:]).astype(jnp.float32)
    p2 = aux2(p_ref[:])

    bit = jax.lax.broadcasted_iota(jnp.int32, (_MB, 128), 0) % 2
    even = bit == 0

    pm0 = jnp.maximum(q0, jnp.where(even, _sub_roll(q0, 1), _sub_roll(q0, -1)))
    pm1 = jnp.maximum(q1, jnp.where(even, _sub_roll(q1, 1), _sub_roll(q1, -1)))
    maxv = jnp.where(b2 == 0.0, pm0, pm1)

    bitf = bit.astype(jnp.float32)
    hit = bitf == b2
    upd0 = q0 + ALPHA * (p2 + GAMMA * maxv - q0)
    upd1 = q1 + ALPHA * (p2 + GAMMA * maxv - q1)
    o_ref[0] = jnp.where((a2 == 0.0) & hit, upd0, q0)
    o_ref[1] = jnp.where((a2 == 1.0) & hit, upd1, q1)


def kernel(type_t_matrix, type_t1_matrix, Q_tensor, profit_matrix):
    n = Q_tensor.shape[0]
    rows = n // 128
    v = (Q_tensor.reshape(rows, 128, 2, 2)
         .transpose(2, 0, 3, 1)
         .reshape(2, 2 * rows, 128))

    out = pl.pallas_call(
        _update_kernel,
        grid=(_L // _BA,),
        in_specs=[
            pl.BlockSpec((_BA, _L), lambda i: (i, 0)),
            pl.BlockSpec((_BA, _L), lambda i: (i, 0)),
            pl.BlockSpec((_BA, _L), lambda i: (i, 0)),
            pl.BlockSpec((2, _MB, 128), lambda i: (0, i, 0)),
        ],
        out_specs=pl.BlockSpec((2, _MB, 128), lambda i: (0, i, 0)),
        out_shape=jax.ShapeDtypeStruct((2, 2 * rows, 128), jnp.float32),
        compiler_params=pltpu.CompilerParams(
            dimension_semantics=("arbitrary",),
        ),
    )(type_t_matrix, type_t1_matrix, profit_matrix, v)

    return (out.reshape(2, rows, 2, 128)
            .transpose(1, 3, 0, 2)
            .reshape(n, 2, 2))


# speedup vs baseline: 184.0429x; 1.1486x over previous
import jax
import jax.numpy as jnp
from jax.experimental import pallas as pl
from jax.experimental.pallas import tpu as pltpu

ALPHA = 0.8
GAMMA = 0.8

_L = 2048
_BA = 16
_CHUNK = _BA * _L
_KB = _CHUNK // 128
_MB = 2 * _KB


def _sub_roll(x, shift):
    return pltpu.roll(x, (-shift) % x.shape[0], axis=0)


def _update_kernel(a_ref, b_ref, p_ref, q_ref, o_ref):
    q = q_ref[:]
    q0 = q[0]
    q1 = q[1]

    ak = a_ref[:].reshape(_KB, 128).astype(jnp.float32)
    bk = b_ref[:].reshape(_KB, 128).astype(jnp.float32)
    pk = p_ref[:].reshape(_KB, 128)
    tk = 2.0 * ak + bk

    def x2(x):
        return jnp.broadcast_to(x[:, None, :], (_KB, 2, 128)).reshape(_MB, 128)

    b2 = x2(bk)
    t2 = x2(tk)
    p2 = x2(pk)

    bit = jax.lax.broadcasted_iota(jnp.int32, (_MB, 128), 0) % 2
    even = bit == 0
    pm0 = jnp.maximum(q0, jnp.where(even, _sub_roll(q0, 1), _sub_roll(q0, -1)))
    pm1 = jnp.maximum(q1, jnp.where(even, _sub_roll(q1, 1), _sub_roll(q1, -1)))
    maxv = jnp.where(b2 == 0.0, pm0, pm1)

    common = ALPHA * p2 + (ALPHA * GAMMA) * maxv
    bitf = bit.astype(jnp.float32)
    w = 1.0 - ALPHA
    o_ref[0] = jnp.where(t2 == bitf, w * q0 + common, q0)
    o_ref[1] = jnp.where(t2 == bitf + 2.0, w * q1 + common, q1)


def kernel(type_t_matrix, type_t1_matrix, Q_tensor, profit_matrix):
    n = Q_tensor.shape[0]
    rows = n // 128
    v = (Q_tensor.reshape(rows, 128, 2, 2)
         .transpose(2, 0, 3, 1)
         .reshape(2, 2 * rows, 128))

    out = pl.pallas_call(
        _update_kernel,
        grid=(_L // _BA,),
        in_specs=[
            pl.BlockSpec((_BA, _L), lambda i: (i, 0)),
            pl.BlockSpec((_BA, _L), lambda i: (i, 0)),
            pl.BlockSpec((_BA, _L), lambda i: (i, 0)),
            pl.BlockSpec((2, _MB, 128), lambda i: (0, i, 0)),
        ],
        out_specs=pl.BlockSpec((2, _MB, 128), lambda i: (0, i, 0)),
        out_shape=jax.ShapeDtypeStruct((2, 2 * rows, 128), jnp.float32),
        compiler_params=pltpu.CompilerParams(
            dimension_semantics=("arbitrary",),
        ),
    )(type_t_matrix, type_t1_matrix, profit_matrix, v)

    return (out.reshape(2, rows, 2, 128)
            .transpose(1, 3, 0, 2)
            .reshape(n, 2, 2))


# speedup vs baseline: 195.0516x; 1.0598x over previous
import jax
import jax.numpy as jnp
from jax.experimental import pallas as pl
from jax.experimental.pallas import tpu as pltpu

ALPHA = 0.8
GAMMA = 0.8

_L = 2048
_BA = 32
_CHUNK = _BA * _L
_KB = _CHUNK // 128
_MB = 2 * _KB


def _sub_roll(x, shift):
    return pltpu.roll(x, (-shift) % x.shape[0], axis=0)


def _update_kernel(a_ref, b_ref, p_ref, q_ref, o_ref):
    q = q_ref[:]
    q0 = q[0]
    q1 = q[1]

    ak = a_ref[:].reshape(_KB, 128).astype(jnp.float32)
    bk = b_ref[:].reshape(_KB, 128).astype(jnp.float32)
    pk = p_ref[:].reshape(_KB, 128)
    tk = 2.0 * ak + bk

    def x2(x):
        return jnp.broadcast_to(x[:, None, :], (_KB, 2, 128)).reshape(_MB, 128)

    b2 = x2(bk)
    t2 = x2(tk)
    p2 = x2(pk)

    bit = jax.lax.broadcasted_iota(jnp.int32, (_MB, 128), 0) % 2
    even = bit == 0
    pm0 = jnp.maximum(q0, jnp.where(even, _sub_roll(q0, 1), _sub_roll(q0, -1)))
    pm1 = jnp.maximum(q1, jnp.where(even, _sub_roll(q1, 1), _sub_roll(q1, -1)))
    maxv = jnp.where(b2 == 0.0, pm0, pm1)

    common = ALPHA * p2 + (ALPHA * GAMMA) * maxv
    bitf = bit.astype(jnp.float32)
    w = 1.0 - ALPHA
    o_ref[0] = jnp.where(t2 == bitf, w * q0 + common, q0)
    o_ref[1] = jnp.where(t2 == bitf + 2.0, w * q1 + common, q1)


def kernel(type_t_matrix, type_t1_matrix, Q_tensor, profit_matrix):
    n = Q_tensor.shape[0]
    rows = n // 128
    v = (Q_tensor.reshape(rows, 128, 2, 2)
         .transpose(2, 0, 3, 1)
         .reshape(2, 2 * rows, 128))

    out = pl.pallas_call(
        _update_kernel,
        grid=(_L // _BA,),
        in_specs=[
            pl.BlockSpec((_BA, _L), lambda i: (i, 0)),
            pl.BlockSpec((_BA, _L), lambda i: (i, 0)),
            pl.BlockSpec((_BA, _L), lambda i: (i, 0)),
            pl.BlockSpec((2, _MB, 128), lambda i: (0, i, 0)),
        ],
        out_specs=pl.BlockSpec((2, _MB, 128), lambda i: (0, i, 0)),
        out_shape=jax.ShapeDtypeStruct((2, 2 * rows, 128), jnp.float32),
        compiler_params=pltpu.CompilerParams(
            dimension_semantics=("parallel",),
        ),
    )(type_t_matrix, type_t1_matrix, profit_matrix, v)

    return (out.reshape(2, rows, 2, 128)
            .transpose(1, 3, 0, 2)
            .reshape(n, 2, 2))


# speedup vs baseline: 195.6383x; 1.0030x over previous
import jax
import jax.numpy as jnp
from jax.experimental import pallas as pl
from jax.experimental.pallas import tpu as pltpu

ALPHA = 0.8
GAMMA = 0.8

_L = 2048
_BA = 64
_CHUNK = _BA * _L
_KB = _CHUNK // 128
_MB = 2 * _KB


def _sub_roll(x, shift):
    return pltpu.roll(x, (-shift) % x.shape[0], axis=0)


def _update_kernel(a_ref, b_ref, p_ref, q_ref, o_ref):
    q = q_ref[:]
    q0 = q[0]
    q1 = q[1]

    ak = a_ref[:].reshape(_KB, 128).astype(jnp.float32)
    bk = b_ref[:].reshape(_KB, 128).astype(jnp.float32)
    pk = p_ref[:].reshape(_KB, 128)
    tk = 2.0 * ak + bk

    def x2(x):
        return jnp.broadcast_to(x[:, None, :], (_KB, 2, 128)).reshape(_MB, 128)

    b2 = x2(bk)
    t2 = x2(tk)
    p2 = x2(pk)

    bit = jax.lax.broadcasted_iota(jnp.int32, (_MB, 128), 0) % 2
    even = bit == 0
    pm0 = jnp.maximum(q0, jnp.where(even, _sub_roll(q0, 1), _sub_roll(q0, -1)))
    pm1 = jnp.maximum(q1, jnp.where(even, _sub_roll(q1, 1), _sub_roll(q1, -1)))
    maxv = jnp.where(b2 == 0.0, pm0, pm1)

    common = ALPHA * p2 + (ALPHA * GAMMA) * maxv
    bitf = bit.astype(jnp.float32)
    w = 1.0 - ALPHA
    o_ref[0] = jnp.where(t2 == bitf, w * q0 + common, q0)
    o_ref[1] = jnp.where(t2 == bitf + 2.0, w * q1 + common, q1)


def kernel(type_t_matrix, type_t1_matrix, Q_tensor, profit_matrix):
    n = Q_tensor.shape[0]
    rows = n // 128
    v = (Q_tensor.reshape(rows, 128, 2, 2)
         .transpose(2, 0, 3, 1)
         .reshape(2, 2 * rows, 128))

    out = pl.pallas_call(
        _update_kernel,
        grid=(_L // _BA,),
        in_specs=[
            pl.BlockSpec((_BA, _L), lambda i: (i, 0)),
            pl.BlockSpec((_BA, _L), lambda i: (i, 0)),
            pl.BlockSpec((_BA, _L), lambda i: (i, 0)),
            pl.BlockSpec((2, _MB, 128), lambda i: (0, i, 0)),
        ],
        out_specs=pl.BlockSpec((2, _MB, 128), lambda i: (0, i, 0)),
        out_shape=jax.ShapeDtypeStruct((2, 2 * rows, 128), jnp.float32),
        compiler_params=pltpu.CompilerParams(
            dimension_semantics=("parallel",),
        ),
    )(type_t_matrix, type_t1_matrix, profit_matrix, v)

    return (out.reshape(2, rows, 2, 128)
            .transpose(1, 3, 0, 2)
            .reshape(n, 2, 2))


# speedup vs baseline: 524.2310x; 2.6796x over previous
import jax
import jax.numpy as jnp
from jax.experimental import pallas as pl
from jax.experimental.pallas import tpu as pltpu

ALPHA = 0.8
GAMMA = 0.8

_L = 2048
_BA = 64
_CHUNK = _BA * _L
_KB = _CHUNK // 128
_MB = 2 * _KB


def _update_kernel(a_ref, b_ref, p_ref, q_ref, o_ref):
    q00 = q_ref[0, 0::2, :]
    q01 = q_ref[0, 1::2, :]
    q10 = q_ref[1, 0::2, :]
    q11 = q_ref[1, 1::2, :]

    ak = a_ref[:].reshape(_KB, 128).astype(jnp.float32)
    bk = b_ref[:].reshape(_KB, 128).astype(jnp.float32)
    pk = p_ref[:].reshape(_KB, 128)
    tk = 2.0 * ak + bk

    m0 = jnp.maximum(q00, q01)
    m1 = jnp.maximum(q10, q11)
    maxv = jnp.where(bk == 0.0, m0, m1)
    common = ALPHA * pk + (ALPHA * GAMMA) * maxv
    w = 1.0 - ALPHA

    o_ref[0, 0::2, :] = jnp.where(tk == 0.0, w * q00 + common, q00)
    o_ref[0, 1::2, :] = jnp.where(tk == 1.0, w * q01 + common, q01)
    o_ref[1, 0::2, :] = jnp.where(tk == 2.0, w * q10 + common, q10)
    o_ref[1, 1::2, :] = jnp.where(tk == 3.0, w * q11 + common, q11)


def kernel(type_t_matrix, type_t1_matrix, Q_tensor, profit_matrix):
    n = Q_tensor.shape[0]
    rows = n // 128
    v = (Q_tensor.reshape(rows, 128, 2, 2)
         .transpose(2, 0, 3, 1)
         .reshape(2, 2 * rows, 128))

    out = pl.pallas_call(
        _update_kernel,
        grid=(_L // _BA,),
        in_specs=[
            pl.BlockSpec((_BA, _L), lambda i: (i, 0)),
            pl.BlockSpec((_BA, _L), lambda i: (i, 0)),
            pl.BlockSpec((_BA, _L), lambda i: (i, 0)),
            pl.BlockSpec((2, _MB, 128), lambda i: (0, i, 0)),
        ],
        out_specs=pl.BlockSpec((2, _MB, 128), lambda i: (0, i, 0)),
        out_shape=jax.ShapeDtypeStruct((2, 2 * rows, 128), jnp.float32),
        compiler_params=pltpu.CompilerParams(
            dimension_semantics=("parallel",),
        ),
    )(type_t_matrix, type_t1_matrix, profit_matrix, v)

    return (out.reshape(2, rows, 2, 128)
            .transpose(1, 3, 0, 2)
            .reshape(n, 2, 2))


# speedup vs baseline: 551.0464x; 1.0512x over previous
import jax
import jax.numpy as jnp
from jax.experimental import pallas as pl
from jax.experimental.pallas import tpu as pltpu

ALPHA = 0.8
GAMMA = 0.8

_L = 2048
_BA = 128
_CHUNK = _BA * _L
_KB = _CHUNK // 128
_MB = 2 * _KB


def _update_kernel(a_ref, b_ref, p_ref, q_ref, o_ref):
    q00 = q_ref[0, 0::2, :]
    q01 = q_ref[0, 1::2, :]
    q10 = q_ref[1, 0::2, :]
    q11 = q_ref[1, 1::2, :]

    ak = a_ref[:].reshape(_KB, 128).astype(jnp.float32)
    bk = b_ref[:].reshape(_KB, 128).astype(jnp.float32)
    pk = p_ref[:].reshape(_KB, 128)
    tk = 2.0 * ak + bk

    m0 = jnp.maximum(q00, q01)
    m1 = jnp.maximum(q10, q11)
    maxv = jnp.where(bk == 0.0, m0, m1)
    common = ALPHA * pk + (ALPHA * GAMMA) * maxv
    w = 1.0 - ALPHA

    o_ref[0, 0::2, :] = jnp.where(tk == 0.0, w * q00 + common, q00)
    o_ref[0, 1::2, :] = jnp.where(tk == 1.0, w * q01 + common, q01)
    o_ref[1, 0::2, :] = jnp.where(tk == 2.0, w * q10 + common, q10)
    o_ref[1, 1::2, :] = jnp.where(tk == 3.0, w * q11 + common, q11)


def kernel(type_t_matrix, type_t1_matrix, Q_tensor, profit_matrix):
    n = Q_tensor.shape[0]
    rows = n // 128
    v = (Q_tensor.reshape(rows, 128, 2, 2)
         .transpose(2, 0, 3, 1)
         .reshape(2, 2 * rows, 128))

    out = pl.pallas_call(
        _update_kernel,
        grid=(_L // _BA,),
        in_specs=[
            pl.BlockSpec((_BA, _L), lambda i: (i, 0)),
            pl.BlockSpec((_BA, _L), lambda i: (i, 0)),
            pl.BlockSpec((_BA, _L), lambda i: (i, 0)),
            pl.BlockSpec((2, _MB, 128), lambda i: (0, i, 0)),
        ],
        out_specs=pl.BlockSpec((2, _MB, 128), lambda i: (0, i, 0)),
        out_shape=jax.ShapeDtypeStruct((2, 2 * rows, 128), jnp.float32),
        compiler_params=pltpu.CompilerParams(
            dimension_semantics=("parallel",),
        ),
    )(type_t_matrix, type_t1_matrix, profit_matrix, v)

    return (out.reshape(2, rows, 2, 128)
            .transpose(1, 3, 0, 2)
            .reshape(n, 2, 2))


# speedup vs baseline: 572.3074x; 1.0386x over previous
import jax
import jax.numpy as jnp
from jax.experimental import pallas as pl
from jax.experimental.pallas import tpu as pltpu

ALPHA = 0.8
GAMMA = 0.8

_L = 2048
_BA = 256
_CHUNK = _BA * _L
_KB = _CHUNK // 128
_MB = 2 * _KB


def _update_kernel(a_ref, b_ref, p_ref, q_ref, o_ref):
    q00 = q_ref[0, 0::2, :]
    q01 = q_ref[0, 1::2, :]
    q10 = q_ref[1, 0::2, :]
    q11 = q_ref[1, 1::2, :]

    ak = a_ref[:].reshape(_KB, 128).astype(jnp.float32)
    bk = b_ref[:].reshape(_KB, 128).astype(jnp.float32)
    pk = p_ref[:].reshape(_KB, 128)
    tk = 2.0 * ak + bk

    m0 = jnp.maximum(q00, q01)
    m1 = jnp.maximum(q10, q11)
    maxv = jnp.where(bk == 0.0, m0, m1)
    common = ALPHA * pk + (ALPHA * GAMMA) * maxv
    w = 1.0 - ALPHA

    o_ref[0, 0::2, :] = jnp.where(tk == 0.0, w * q00 + common, q00)
    o_ref[0, 1::2, :] = jnp.where(tk == 1.0, w * q01 + common, q01)
    o_ref[1, 0::2, :] = jnp.where(tk == 2.0, w * q10 + common, q10)
    o_ref[1, 1::2, :] = jnp.where(tk == 3.0, w * q11 + common, q11)


def kernel(type_t_matrix, type_t1_matrix, Q_tensor, profit_matrix):
    n = Q_tensor.shape[0]
    rows = n // 128
    v = (Q_tensor.reshape(rows, 128, 2, 2)
         .transpose(2, 0, 3, 1)
         .reshape(2, 2 * rows, 128))

    out = pl.pallas_call(
        _update_kernel,
        grid=(_L // _BA,),
        in_specs=[
            pl.BlockSpec((_BA, _L), lambda i: (i, 0)),
            pl.BlockSpec((_BA, _L), lambda i: (i, 0)),
            pl.BlockSpec((_BA, _L), lambda i: (i, 0)),
            pl.BlockSpec((2, _MB, 128), lambda i: (0, i, 0)),
        ],
        out_specs=pl.BlockSpec((2, _MB, 128), lambda i: (0, i, 0)),
        out_shape=jax.ShapeDtypeStruct((2, 2 * rows, 128), jnp.float32),
        compiler_params=pltpu.CompilerParams(
            dimension_semantics=("parallel",),
        ),
    )(type_t_matrix, type_t1_matrix, profit_matrix, v)

    return (out.reshape(2, rows, 2, 128)
            .transpose(1, 3, 0, 2)
            .reshape(n, 2, 2))
